# baseline (speedup 1.0000x reference)
import jax
import jax.numpy as jnp
from jax import lax
from jax.experimental import pallas as pl

N_USERS = 4096
NUM_SUBREDDITS = 20000
X_DIM = 64
SUB_REP_DIM = 3

BM = 256
CHUNK = 128
NFULL = NUM_SUBREDDITS // CHUNK
REM = NUM_SUBREDDITS - NFULL * CHUNK


def _agg_kernel(x_ref, s_ref, rt_ref, o_ref):
    def body(i, accs):
        a0, a1, a2 = accs
        s = s_ref[:, pl.ds(i * CHUNK, CHUNK)]
        r = rt_ref[:, pl.ds(i * CHUNK, CHUNK)]
        a0 = a0 + s * r[0:1, :]
        a1 = a1 + s * r[1:2, :]
        a2 = a2 + s * r[2:3, :]
        return a0, a1, a2

    zero = jnp.zeros((BM, CHUNK), jnp.float32)
    a0, a1, a2 = lax.fori_loop(0, NFULL, body, (zero, zero, zero))

    s = s_ref[:, NFULL * CHUNK:]
    r = rt_ref[:, NFULL * CHUNK:]
    c0 = jnp.sum(a0, axis=1, keepdims=True) + jnp.sum(s * r[0:1, :], axis=1, keepdims=True)
    c1 = jnp.sum(a1, axis=1, keepdims=True) + jnp.sum(s * r[1:2, :], axis=1, keepdims=True)
    c2 = jnp.sum(a2, axis=1, keepdims=True) + jnp.sum(s * r[2:3, :], axis=1, keepdims=True)

    o_ref[:, :X_DIM] = x_ref[...]
    o_ref[:, X_DIM:] = jnp.concatenate([c0, c1, c2], axis=1)


def kernel(x, S, R):
    grid = (N_USERS // BM,)
    out = pl.pallas_call(
        _agg_kernel,
        grid=grid,
        in_specs=[
            pl.BlockSpec((BM, X_DIM), lambda i: (i, 0)),
            pl.BlockSpec((BM, NUM_SUBREDDITS), lambda i: (i, 0)),
            pl.BlockSpec((SUB_REP_DIM, NUM_SUBREDDITS), lambda i: (0, 0)),
        ],
        out_specs=pl.BlockSpec((BM, X_DIM + SUB_REP_DIM), lambda i: (i, 0)),
        out_shape=jax.ShapeDtypeStruct((N_USERS, X_DIM + SUB_REP_DIM), jnp.float32),
    )(x, S, R.T)
    return out


# speedup vs baseline: 1.2649x; 1.2649x over previous
import jax
import jax.numpy as jnp
from jax import lax
from jax.experimental import pallas as pl

N_USERS = 4096
NUM_SUBREDDITS = 20000
X_DIM = 64
SUB_REP_DIM = 3

BM = 256


def _agg_kernel(x_ref, s_ref, rt_ref, o_ref):
    s = s_ref[...].astype(jnp.bfloat16)
    rt = rt_ref[...].astype(jnp.bfloat16)
    acc_t = lax.dot_general(
        rt, s,
        dimension_numbers=(((1,), (1,)), ((), ())),
        preferred_element_type=jnp.float32,
    )
    o_ref[:, :X_DIM] = x_ref[...]
    o_ref[:, X_DIM:] = acc_t.T


def kernel(x, S, R):
    grid = (N_USERS // BM,)
    out = pl.pallas_call(
        _agg_kernel,
        grid=grid,
        in_specs=[
            pl.BlockSpec((BM, X_DIM), lambda i: (i, 0)),
            pl.BlockSpec((BM, NUM_SUBREDDITS), lambda i: (i, 0)),
            pl.BlockSpec((SUB_REP_DIM, NUM_SUBREDDITS), lambda i: (0, 0)),
        ],
        out_specs=pl.BlockSpec((BM, X_DIM + SUB_REP_DIM), lambda i: (i, 0)),
        out_shape=jax.ShapeDtypeStruct((N_USERS, X_DIM + SUB_REP_DIM), jnp.float32),
    )(x, S, R.T)
    return out


# speedup vs baseline: 1.2688x; 1.0031x over previous
import jax
import jax.numpy as jnp
from jax import lax
from jax.experimental import pallas as pl

N_USERS = 4096
NUM_SUBREDDITS = 20000
X_DIM = 64
SUB_REP_DIM = 3

NSTREAM = 4
BMS = 64
BM = NSTREAM * BMS


def _agg_kernel(x_ref, *refs):
    s_refs = refs[:NSTREAM]
    rt_ref = refs[NSTREAM]
    o_ref = refs[NSTREAM + 1]
    rt = rt_ref[...].astype(jnp.bfloat16)
    o_ref[:, :X_DIM] = x_ref[...]
    for k in range(NSTREAM):
        s = s_refs[k][...].astype(jnp.bfloat16)
        acc_t = lax.dot_general(
            rt, s,
            dimension_numbers=(((1,), (1,)), ((), ())),
            preferred_element_type=jnp.float32,
        )
        o_ref[k * BMS:(k + 1) * BMS, X_DIM:] = acc_t.T


def kernel(x, S, R):
    grid = (N_USERS // BM,)
    s_specs = [
        pl.BlockSpec((BMS, NUM_SUBREDDITS), lambda i, k=k: (i * NSTREAM + k, 0))
        for k in range(NSTREAM)
    ]
    out = pl.pallas_call(
        _agg_kernel,
        grid=grid,
        in_specs=[pl.BlockSpec((BM, X_DIM), lambda i: (i, 0))]
        + s_specs
        + [pl.BlockSpec((SUB_REP_DIM, NUM_SUBREDDITS), lambda i: (0, 0))],
        out_specs=pl.BlockSpec((BM, X_DIM + SUB_REP_DIM), lambda i: (i, 0)),
        out_shape=jax.ShapeDtypeStruct((N_USERS, X_DIM + SUB_REP_DIM), jnp.float32),
    )(x, *([S] * NSTREAM), R.T)
    return out


# speedup vs baseline: 1.3350x; 1.0522x over previous
import jax
import jax.numpy as jnp
from jax import lax
from jax.experimental import pallas as pl
from jax.experimental.pallas import tpu as pltpu

N_USERS = 4096
NUM_SUBREDDITS = 20000
X_DIM = 64
SUB_REP_DIM = 3

NSTREAM = 8
BM = 256
BMS = BM // NSTREAM


def _agg_kernel(x_ref, s_hbm, rt_ref, o_ref, buf, sems):
    i = pl.program_id(0)
    nsteps = pl.num_programs(0)

    def start(step, slot):
        for k in range(NSTREAM):
            pltpu.make_async_copy(
                s_hbm.at[pl.ds(step * BM + k * BMS, BMS), :],
                buf.at[slot, pl.ds(k * BMS, BMS), :],
                sems.at[slot, k],
            ).start()

    @pl.when(i == 0)
    def _():
        start(0, 0)

    @pl.when(i + 1 < nsteps)
    def _():
        start(i + 1, (i + 1) % 2)

    slot = i % 2
    for k in range(NSTREAM):
        pltpu.make_async_copy(
            s_hbm.at[pl.ds(i * BM + k * BMS, BMS), :],
            buf.at[slot, pl.ds(k * BMS, BMS), :],
            sems.at[slot, k],
        ).wait()

    s = buf[slot].astype(jnp.bfloat16)
    rt = rt_ref[...].astype(jnp.bfloat16)
    acc_t = lax.dot_general(
        rt, s,
        dimension_numbers=(((1,), (1,)), ((), ())),
        preferred_element_type=jnp.float32,
    )
    o_ref[:, :X_DIM] = x_ref[...]
    o_ref[:, X_DIM:] = acc_t.T


def kernel(x, S, R):
    grid = (N_USERS // BM,)
    out = pl.pallas_call(
        _agg_kernel,
        grid=grid,
        in_specs=[
            pl.BlockSpec((BM, X_DIM), lambda i: (i, 0)),
            pl.BlockSpec(memory_space=pltpu.MemorySpace.HBM),
            pl.BlockSpec((SUB_REP_DIM, NUM_SUBREDDITS), lambda i: (0, 0)),
        ],
        out_specs=pl.BlockSpec((BM, X_DIM + SUB_REP_DIM), lambda i: (i, 0)),
        out_shape=jax.ShapeDtypeStruct((N_USERS, X_DIM + SUB_REP_DIM), jnp.float32),
        scratch_shapes=[
            pltpu.VMEM((2, BM, NUM_SUBREDDITS), jnp.float32),
            pltpu.SemaphoreType.DMA((2, NSTREAM)),
        ],
        compiler_params=pltpu.CompilerParams(
            dimension_semantics=("arbitrary",),
            vmem_limit_bytes=100 * 1024 * 1024,
        ),
    )(x, S, R.T)
    return out
